# baseline (speedup 1.0000x reference)
import functools

import jax
import jax.numpy as jnp
from jax import lax
from jax.experimental import pallas as pl
from jax.experimental.pallas import tpu as pltpu
from jax.experimental.pallas import tpu_sc as plsc

_CHANNELS = 64
_N_TOKENS = 16384
_LANES = 16


def _build_sc_kernel(B, D, C, NC, NS):
    NW = NC * NS
    per_w = B // NW
    n_chunks = per_w // C
    mesh = plsc.VectorSubcoreMesh(core_axis_name="c", subcore_axis_name="s")

    @functools.partial(
        pl.kernel,
        mesh=mesh,
        out_type=jax.ShapeDtypeStruct((3, B, D), jnp.float32),
        compiler_params=pltpu.CompilerParams(use_tc_tiling_on_sc=False),
        scratch_types=[
            pltpu.VMEM((C,), jnp.int32),
            pltpu.VMEM((C,), jnp.int32),
            pltpu.VMEM((C,), jnp.int32),
            pltpu.VMEM((C,), jnp.int32),
            pltpu.VMEM((C, D), jnp.float32),
            pltpu.VMEM((C, D), jnp.float32),
            pltpu.VMEM((C, D), jnp.float32),
            pltpu.VMEM((C, D), jnp.float32),
            pltpu.SemaphoreType.DMA,
        ],
    )
    def k(xa, xm, xr, xt, wa, wm, wr, wt, out, ia, im, ir, it, A, M, R, T, sem):
        wid = lax.axis_index("s") * NC + lax.axis_index("c")
        base0 = wid * per_w

        def chunk(ci, _):
            base = base0 + ci * C
            pltpu.sync_copy(xa.at[pl.ds(base, C)], ia)
            pltpu.sync_copy(xm.at[pl.ds(base, C)], im)
            pltpu.sync_copy(xr.at[pl.ds(base, C)], ir)
            pltpu.sync_copy(xt.at[pl.ds(base, C)], it)
            cps = [
                pltpu.async_copy(wa.at[ia], A, sem),
                pltpu.async_copy(wm.at[im], M, sem),
                pltpu.async_copy(wr.at[ir], R, sem),
                pltpu.async_copy(wt.at[it], T, sem),
            ]
            for cp in cps:
                cp.wait()

            def row(i, _2):
                for j in range(D // _LANES):
                    sl = pl.ds(j * _LANES, _LANES)
                    tv = T[i, sl]
                    A[i, sl] = A[i, sl] + tv
                    M[i, sl] = M[i, sl] + tv
                    R[i, sl] = R[i, sl] + tv
                return 0

            lax.fori_loop(0, C, row, 0)
            pltpu.sync_copy(A, out.at[0, pl.ds(base, C)])
            pltpu.sync_copy(M, out.at[1, pl.ds(base, C)])
            pltpu.sync_copy(R, out.at[2, pl.ds(base, C)])
            return 0

        lax.fori_loop(0, n_chunks, chunk, 0)

    return k


def kernel(x_action, x_mode, x_readout, t, W_action, W_mode, W_readout, W_time):
    info = plsc.get_sparse_core_info()
    k = _build_sc_kernel(_N_TOKENS, _CHANNELS, 128, info.num_cores,
                         info.num_subcores)
    return k(x_action.astype(jnp.int32), x_mode.astype(jnp.int32),
             x_readout.astype(jnp.int32), t.astype(jnp.int32),
             W_action, W_mode, W_readout, W_time)
